# Initial kernel scaffold; baseline (speedup 1.0000x reference)
import jax
import jax.numpy as jnp
from jax.experimental import pallas as pl


def kernel(page_hash, offset, cache_line, addr_alignment, stride, reuse_dist, locality_cluster, entropy, address_flags, E_ph, E_off, E_cl, E_aa, E_str, E_rd, E_lc, E_ent, W_flags, b_flags, W_gate, b_gate, W_proj, b_proj, gamma, beta):
    raise NotImplementedError("write your pallas kernel here")


# speedup vs baseline: 5.5304x; 5.5304x over previous
import functools

import jax
import jax.numpy as jnp
from jax import lax
from jax.experimental import pallas as pl
from jax.experimental.pallas import tpu as pltpu
from jax.experimental.pallas import tpu_sc as plsc

B, L = 4096, 50
N = B * L
D_OUT = 128
EMB_DIMS = (8, 4, 4, 3, 6, 5, 6, 3)
D_EMB = 39
D_PAD = 64

NC, NS = 2, 16
NW = NC * NS
PER_W = N // NW
CHUNK = 128
NCHUNK = PER_W // CHUNK
R_CHUNK = CHUNK * 8


def _sc_gather_stacked(idx8, table):
    mesh = plsc.VectorSubcoreMesh(core_axis_name="c", subcore_axis_name="s")

    @functools.partial(
        pl.kernel,
        out_type=jax.ShapeDtypeStruct((8 * N, 8), jnp.float32),
        mesh=mesh,
        scratch_types=[pltpu.VMEM((R_CHUNK,), jnp.int32),
                       pltpu.VMEM((R_CHUNK, 8), jnp.float32),
                       pltpu.SemaphoreType.DMA],
        compiler_params=pltpu.CompilerParams(use_tc_tiling_on_sc=False),
    )
    def body(idx_hbm, tab_hbm, out_hbm, idx_v, gat_v, sem):
        wid = lax.axis_index("s") * NC + lax.axis_index("c")
        base = wid * PER_W * 8

        def chunk_step(ci, carry):
            row = base + ci * R_CHUNK
            pltpu.sync_copy(idx_hbm.at[pl.ds(row, R_CHUNK)], idx_v)
            pltpu.async_copy(tab_hbm.at[idx_v], gat_v, sem).wait()
            pltpu.sync_copy(gat_v, out_hbm.at[pl.ds(row, R_CHUNK), :])
            return carry

        lax.fori_loop(0, NCHUNK, chunk_step, 0)

    return body(idx8, table)


TB = 2048


def _tc_body(comb_ref, flg_ref, wg_ref, wfg_ref, bg_ref,
             wp_ref, wfp_ref, bp_ref, gam_ref, bet_ref, out_ref):
    c = comb_ref[:]
    f = flg_ref[:]
    hi = lax.Precision.HIGHEST
    a = (jnp.dot(c, wg_ref[:], preferred_element_type=jnp.float32, precision=hi)
         + jnp.dot(f, wfg_ref[:], preferred_element_type=jnp.float32,
                   precision=hi) + bg_ref[:])
    gate = jax.nn.sigmoid(a)
    h = (jnp.dot(c, wp_ref[:], preferred_element_type=jnp.float32, precision=hi)
         + jnp.dot(f, wfp_ref[:], preferred_element_type=jnp.float32,
                   precision=hi) + bp_ref[:])
    z = gate * h
    mu = jnp.mean(z, axis=1, keepdims=True)
    zc = z - mu
    var = jnp.mean(zc * zc, axis=1, keepdims=True)
    out_ref[:] = zc * lax.rsqrt(var + 1e-5) * gam_ref[:] + bet_ref[:]


def _tc_fuse(comb, flags2d, wg, wfg, bg, wp, wfp, bp, gamma, beta):
    grid = (N // TB,)
    full = lambda shape: pl.BlockSpec(shape, lambda i: (0, 0))
    return pl.pallas_call(
        _tc_body,
        grid=grid,
        in_specs=[
            pl.BlockSpec((TB, D_PAD), lambda i: (i, 0)),
            pl.BlockSpec((TB, 5), lambda i: (i, 0)),
            full((D_PAD, D_OUT)), full((5, D_OUT)), full((1, D_OUT)),
            full((D_PAD, D_OUT)), full((5, D_OUT)), full((1, D_OUT)),
            full((1, D_OUT)), full((1, D_OUT)),
        ],
        out_specs=pl.BlockSpec((TB, D_OUT), lambda i: (i, 0)),
        out_shape=jax.ShapeDtypeStruct((N, D_OUT), jnp.float32),
    )(comb, flags2d, wg, wfg, bg, wp, wfp, bp, gamma, beta)


def _pad_weight(W):
    parts = []
    off = 0
    for d in EMB_DIMS:
        parts.append(W[off:off + d])
        if d < 8:
            parts.append(jnp.zeros((8 - d, D_OUT), W.dtype))
        off += d
    return jnp.concatenate(parts, axis=0)


def _pad_table(E, d):
    if d == 8:
        return E
    return jnp.pad(E, ((0, 0), (0, 8 - d)))


def kernel(page_hash, offset, cache_line, addr_alignment, stride, reuse_dist,
           locality_cluster, entropy, address_flags,
           E_ph, E_off, E_cl, E_aa, E_str, E_rd, E_lc, E_ent,
           W_flags, b_flags, W_gate, b_gate, W_proj, b_proj, gamma, beta):
    idx_raw = (page_hash, offset, cache_line, addr_alignment, stride,
               reuse_dist, locality_cluster, entropy)
    tables = (E_ph, E_off, E_cl, E_aa, E_str, E_rd, E_lc, E_ent)
    flags2d = address_flags.reshape(N, 5)

    stacked = jnp.concatenate(
        [_pad_table(E, d) for E, d in zip(tables, EMB_DIMS)], axis=0)
    bases = []
    acc = 0
    for E in tables:
        bases.append(acc)
        acc += E.shape[0]
    idx8 = jnp.stack(
        [x.reshape(N) + b for x, b in zip(idx_raw, bases)],
        axis=1).reshape(8 * N)

    scale = jnp.mean(jnp.abs(W_flags)) + 1e-8
    Wq = jnp.clip(jnp.round(W_flags / scale), -1.0, 1.0) * scale
    wg = _pad_weight(W_gate[:D_EMB])
    wfg = Wq.T @ W_gate[D_EMB:]
    bg = (b_gate + b_flags @ W_gate[D_EMB:]).reshape(1, D_OUT)
    wp = _pad_weight(W_proj[:D_EMB])
    wfp = Wq.T @ W_proj[D_EMB:]
    bp = (b_proj + b_flags @ W_proj[D_EMB:]).reshape(1, D_OUT)

    comb = _sc_gather_stacked(idx8, stacked).reshape(N, D_PAD)
    out = _tc_fuse(comb, flags2d, wg, wfg, bg, wp, wfp, bp,
                   gamma.reshape(1, D_OUT), beta.reshape(1, D_OUT))
    return out.reshape(B, L, D_OUT)


# speedup vs baseline: 6.4726x; 1.1704x over previous
import functools

import jax
import jax.numpy as jnp
from jax import lax
from jax.experimental import pallas as pl
from jax.experimental.pallas import tpu as pltpu
from jax.experimental.pallas import tpu_sc as plsc

B, L = 4096, 50
N = B * L
D_OUT = 128
EMB_DIMS = (8, 4, 4, 3, 6, 5, 6, 3)
DIMS7 = EMB_DIMS[1:]
D_EMB = 39
D7 = 56

NC, NS = 2, 16
NW = NC * NS
PER_W = N // NW
CHUNK = 640
NCHUNK = PER_W // CHUNK
R7 = CHUNK * 7


def _sc_gather2(idx_ph, idx7, E_ph, tab7):
    mesh = plsc.VectorSubcoreMesh(core_axis_name="c", subcore_axis_name="s")

    scratch = []
    for _ in range(2):
        scratch += [pltpu.VMEM((CHUNK,), jnp.int32),
                    pltpu.VMEM((R7,), jnp.int32),
                    pltpu.VMEM((CHUNK, 8), jnp.float32),
                    pltpu.VMEM((R7, 8), jnp.float32)]
    scratch += [pltpu.SemaphoreType.DMA] * 12

    @functools.partial(
        pl.kernel,
        out_type=(jax.ShapeDtypeStruct((N, 8), jnp.float32),
                  jax.ShapeDtypeStruct((7 * N, 8), jnp.float32)),
        mesh=mesh,
        scratch_types=scratch,
        compiler_params=pltpu.CompilerParams(use_tc_tiling_on_sc=False),
    )
    def body(iph_hbm, i7_hbm, eph_hbm, t7_hbm, oph_hbm, o7_hbm,
             xph0, x70, gph0, g70, xph1, x71, gph1, g71,
             sxp0, sx70, sgp0, sg70, sop0, so70,
             sxp1, sx71, sgp1, sg71, sop1, so71):
        xph = (xph0, xph1)
        x7 = (x70, x71)
        gph = (gph0, gph1)
        g7 = (g70, g71)
        s_xph = (sxp0, sxp1)
        s_x7 = (sx70, sx71)
        s_gph = (sgp0, sgp1)
        s_g7 = (sg70, sg71)
        s_oph = (sop0, sop1)
        s_o7 = (so70, so71)

        wid = lax.axis_index("s") * NC + lax.axis_index("c")
        tok0 = wid * PER_W

        def start_idx(ci, X):
            t = tok0 + ci * CHUNK
            return (
                pltpu.async_copy(iph_hbm.at[pl.ds(t, CHUNK)], xph[X], s_xph[X]),
                pltpu.async_copy(i7_hbm.at[pl.ds(t * 7, R7)], x7[X], s_x7[X]),
            )

        pend_idx = [start_idx(0, 0),
                    start_idx(1, 1) if NCHUNK > 1 else None]
        pend_out = [None, None]

        for ci in range(NCHUNK):
            X = ci % 2
            t = tok0 + ci * CHUNK
            for d in pend_idx[X]:
                d.wait()
            if pend_out[X] is not None:
                for d in pend_out[X]:
                    d.wait()
            dg1 = pltpu.async_copy(eph_hbm.at[xph[X]], gph[X], s_gph[X])
            dg2 = pltpu.async_copy(t7_hbm.at[x7[X]], g7[X], s_g7[X])
            dg1.wait()
            dg2.wait()
            if ci + 2 < NCHUNK:
                pend_idx[X] = start_idx(ci + 2, X)
            pend_out[X] = (
                pltpu.async_copy(gph[X], oph_hbm.at[pl.ds(t, CHUNK), :],
                                 s_oph[X]),
                pltpu.async_copy(g7[X], o7_hbm.at[pl.ds(t * 7, R7), :],
                                 s_o7[X]),
            )

        for p in pend_out:
            if p is not None:
                for d in p:
                    d.wait()

    return body(idx_ph, idx7, E_ph, tab7)


TB = 2048


def _tc_body(ph_ref, c7_ref, flg_ref,
             wgp_ref, wg7_ref, wgf_ref, bg_ref,
             wpp_ref, wp7_ref, wpf_ref, bp_ref,
             gam_ref, bet_ref, out_ref):
    ph = ph_ref[:]
    c7 = c7_ref[:]
    f = flg_ref[:]

    def mm(x, w):
        return jnp.dot(x, w[:], preferred_element_type=jnp.float32)

    a = mm(ph, wgp_ref) + mm(c7, wg7_ref) + mm(f, wgf_ref) + bg_ref[:]
    gate = jax.nn.sigmoid(a)
    h = mm(ph, wpp_ref) + mm(c7, wp7_ref) + mm(f, wpf_ref) + bp_ref[:]
    z = gate * h
    mu = jnp.mean(z, axis=1, keepdims=True)
    zc = z - mu
    var = jnp.mean(zc * zc, axis=1, keepdims=True)
    out_ref[:] = zc * lax.rsqrt(var + 1e-5) * gam_ref[:] + bet_ref[:]


def _tc_fuse(ph, c7, flags2d, wgp, wg7, wgf, bg, wpp, wp7, wpf, bp,
             gamma, beta):
    grid = (N // TB,)
    full = lambda shape: pl.BlockSpec(shape, lambda i: (0, 0))
    return pl.pallas_call(
        _tc_body,
        grid=grid,
        in_specs=[
            pl.BlockSpec((TB, 8), lambda i: (i, 0)),
            pl.BlockSpec((TB, D7), lambda i: (i, 0)),
            pl.BlockSpec((TB, 5), lambda i: (i, 0)),
            full((8, D_OUT)), full((D7, D_OUT)), full((5, D_OUT)),
            full((1, D_OUT)),
            full((8, D_OUT)), full((D7, D_OUT)), full((5, D_OUT)),
            full((1, D_OUT)),
            full((1, D_OUT)), full((1, D_OUT)),
        ],
        out_specs=pl.BlockSpec((TB, D_OUT), lambda i: (i, 0)),
        out_shape=jax.ShapeDtypeStruct((N, D_OUT), jnp.float32),
    )(ph, c7, flags2d, wgp, wg7, wgf, bg, wpp, wp7, wpf, bp, gamma, beta)


def _pad_weight7(W31):
    parts = []
    off = 0
    for d in DIMS7:
        parts.append(W31[off:off + d])
        if d < 8:
            parts.append(jnp.zeros((8 - d, D_OUT), W31.dtype))
        off += d
    return jnp.concatenate(parts, axis=0)


def kernel(page_hash, offset, cache_line, addr_alignment, stride, reuse_dist,
           locality_cluster, entropy, address_flags,
           E_ph, E_off, E_cl, E_aa, E_str, E_rd, E_lc, E_ent,
           W_flags, b_flags, W_gate, b_gate, W_proj, b_proj, gamma, beta):
    idx7_raw = (offset, cache_line, addr_alignment, stride,
                reuse_dist, locality_cluster, entropy)
    tables7 = (E_off, E_cl, E_aa, E_str, E_rd, E_lc, E_ent)
    flags2d = address_flags.reshape(N, 5)

    tab7 = jnp.concatenate(
        [E if d == 8 else jnp.pad(E, ((0, 0), (0, 8 - d)))
         for E, d in zip(tables7, DIMS7)], axis=0)
    bases = []
    acc = 0
    for E in tables7:
        bases.append(acc)
        acc += E.shape[0]
    idx7 = jnp.stack(
        [x.reshape(N) + b for x, b in zip(idx7_raw, bases)],
        axis=1).reshape(7 * N)
    idx_ph = page_hash.reshape(N)

    scale = jnp.mean(jnp.abs(W_flags)) + 1e-8
    Wq = jnp.clip(jnp.round(W_flags / scale), -1.0, 1.0) * scale
    wgp, wg7 = W_gate[:8], _pad_weight7(W_gate[8:D_EMB])
    wgf = Wq.T @ W_gate[D_EMB:]
    bg = (b_gate + b_flags @ W_gate[D_EMB:]).reshape(1, D_OUT)
    wpp, wp7 = W_proj[:8], _pad_weight7(W_proj[8:D_EMB])
    wpf = Wq.T @ W_proj[D_EMB:]
    bp = (b_proj + b_flags @ W_proj[D_EMB:]).reshape(1, D_OUT)

    ph, c7 = _sc_gather2(idx_ph, idx7, E_ph, tab7)
    out = _tc_fuse(ph, c7.reshape(N, D7), flags2d,
                   wgp, wg7, wgf, bg, wpp, wp7, wpf, bp,
                   gamma.reshape(1, D_OUT), beta.reshape(1, D_OUT))
    return out.reshape(B, L, D_OUT)


# speedup vs baseline: 6.9204x; 1.0692x over previous
import functools

import jax
import jax.numpy as jnp
from jax import lax
from jax.experimental import pallas as pl
from jax.experimental.pallas import tpu as pltpu
from jax.experimental.pallas import tpu_sc as plsc

B, L = 4096, 50
N = B * L
D_OUT = 128
EMB_DIMS = (8, 4, 4, 3, 6, 5, 6, 3)
DIMS7 = EMB_DIMS[1:]
D_EMB = 39
D7 = 56

NC, NS = 2, 16
NW = NC * NS
PER_W = N // NW
CHUNK = 640
NCHUNK = PER_W // CHUNK
R7 = CHUNK * 7


def _sc_gather2(idx_ph, idx7, E_ph, tab7):
    mesh = plsc.VectorSubcoreMesh(core_axis_name="c", subcore_axis_name="s")

    scratch = []
    for _ in range(2):
        scratch += [pltpu.VMEM((CHUNK,), jnp.int32),
                    pltpu.VMEM((R7,), jnp.int32),
                    pltpu.VMEM((CHUNK, 8), jnp.float32),
                    pltpu.VMEM((R7, 8), jnp.float32)]
    scratch += [pltpu.SemaphoreType.DMA] * 12

    @functools.partial(
        pl.kernel,
        out_type=(jax.ShapeDtypeStruct((N, 8), jnp.float32),
                  jax.ShapeDtypeStruct((7 * N, 8), jnp.float32)),
        mesh=mesh,
        scratch_types=scratch,
        compiler_params=pltpu.CompilerParams(use_tc_tiling_on_sc=False),
    )
    def body(iph_hbm, i7_hbm, eph_hbm, t7_hbm, oph_hbm, o7_hbm,
             xph0, x70, gph0, g70, xph1, x71, gph1, g71,
             sxp0, sx70, sgp0, sg70, sop0, so70,
             sxp1, sx71, sgp1, sg71, sop1, so71):
        xph = (xph0, xph1)
        x7 = (x70, x71)
        gph = (gph0, gph1)
        g7 = (g70, g71)
        s_xph = (sxp0, sxp1)
        s_x7 = (sx70, sx71)
        s_gph = (sgp0, sgp1)
        s_g7 = (sg70, sg71)
        s_oph = (sop0, sop1)
        s_o7 = (so70, so71)

        wid = lax.axis_index("s") * NC + lax.axis_index("c")
        tok0 = wid * PER_W

        def start_idx(ci, X):
            t = tok0 + ci * CHUNK
            return (
                pltpu.async_copy(iph_hbm.at[pl.ds(t, CHUNK)], xph[X], s_xph[X]),
                pltpu.async_copy(i7_hbm.at[pl.ds(t * 7, R7)], x7[X], s_x7[X]),
            )

        pend_idx = [start_idx(0, 0),
                    start_idx(1, 1) if NCHUNK > 1 else None]
        pend_out = [None, None]

        for ci in range(NCHUNK):
            X = ci % 2
            t = tok0 + ci * CHUNK
            for d in pend_idx[X]:
                d.wait()
            if pend_out[X] is not None:
                for d in pend_out[X]:
                    d.wait()
            dg1 = pltpu.async_copy(eph_hbm.at[xph[X]], gph[X], s_gph[X])
            dg2 = pltpu.async_copy(t7_hbm.at[x7[X]], g7[X], s_g7[X])
            dg1.wait()
            dg2.wait()
            if ci + 2 < NCHUNK:
                pend_idx[X] = start_idx(ci + 2, X)
            pend_out[X] = (
                pltpu.async_copy(gph[X], oph_hbm.at[pl.ds(t, CHUNK), :],
                                 s_oph[X]),
                pltpu.async_copy(g7[X], o7_hbm.at[pl.ds(t * 7, R7), :],
                                 s_o7[X]),
            )

        for p in pend_out:
            if p is not None:
                for d in p:
                    d.wait()

    return body(idx_ph, idx7, E_ph, tab7)


TB = 1600
BBLK = TB // L


def _tc_body(ph_ref, c7_ref, flg_ref,
             wgp_ref, wg7_ref, wgf_ref, bg_ref,
             wpp_ref, wp7_ref, wpf_ref, bp_ref,
             gam_ref, bet_ref, out_ref):
    ph = ph_ref[:]
    c7 = c7_ref[:]
    f = flg_ref[:]

    def mm(x, w):
        return jnp.dot(x, w[:], preferred_element_type=jnp.float32)

    a = mm(ph, wgp_ref) + mm(c7, wg7_ref) + mm(f, wgf_ref) + bg_ref[:]
    gate = jax.nn.sigmoid(a)
    h = mm(ph, wpp_ref) + mm(c7, wp7_ref) + mm(f, wpf_ref) + bp_ref[:]
    z = gate * h
    mu = jnp.mean(z, axis=1, keepdims=True)
    zc = z - mu
    var = jnp.mean(zc * zc, axis=1, keepdims=True)
    out_ref[:] = (zc * lax.rsqrt(var + 1e-5) * gam_ref[:]
                  + bet_ref[:]).reshape(BBLK, L, D_OUT)


def _tc_fuse(ph, c7, flags2d, wgp, wg7, wgf, bg, wpp, wp7, wpf, bp,
             gamma, beta):
    grid = (N // TB,)
    full = lambda shape: pl.BlockSpec(shape, lambda i: (0, 0))
    return pl.pallas_call(
        _tc_body,
        grid=grid,
        in_specs=[
            pl.BlockSpec((TB, 8), lambda i: (i, 0)),
            pl.BlockSpec((TB, D7), lambda i: (i, 0)),
            pl.BlockSpec((TB, 5), lambda i: (i, 0)),
            full((8, D_OUT)), full((D7, D_OUT)), full((5, D_OUT)),
            full((1, D_OUT)),
            full((8, D_OUT)), full((D7, D_OUT)), full((5, D_OUT)),
            full((1, D_OUT)),
            full((1, D_OUT)), full((1, D_OUT)),
        ],
        out_specs=pl.BlockSpec((BBLK, L, D_OUT), lambda i: (i, 0, 0)),
        out_shape=jax.ShapeDtypeStruct((B, L, D_OUT), jnp.float32),
    )(ph, c7, flags2d, wgp, wg7, wgf, bg, wpp, wp7, wpf, bp, gamma, beta)


def _pad_weight7(W31):
    parts = []
    off = 0
    for d in DIMS7:
        parts.append(W31[off:off + d])
        if d < 8:
            parts.append(jnp.zeros((8 - d, D_OUT), W31.dtype))
        off += d
    return jnp.concatenate(parts, axis=0)


def kernel(page_hash, offset, cache_line, addr_alignment, stride, reuse_dist,
           locality_cluster, entropy, address_flags,
           E_ph, E_off, E_cl, E_aa, E_str, E_rd, E_lc, E_ent,
           W_flags, b_flags, W_gate, b_gate, W_proj, b_proj, gamma, beta):
    idx7_raw = (offset, cache_line, addr_alignment, stride,
                reuse_dist, locality_cluster, entropy)
    tables7 = (E_off, E_cl, E_aa, E_str, E_rd, E_lc, E_ent)
    flags2d = address_flags.reshape(N, 5)

    tab7 = jnp.concatenate(
        [E if d == 8 else jnp.pad(E, ((0, 0), (0, 8 - d)))
         for E, d in zip(tables7, DIMS7)], axis=0)
    bases = []
    acc = 0
    for E in tables7:
        bases.append(acc)
        acc += E.shape[0]
    idx7 = jnp.stack(
        [x.reshape(N) + b for x, b in zip(idx7_raw, bases)],
        axis=1).reshape(7 * N)
    idx_ph = page_hash.reshape(N)

    scale = jnp.mean(jnp.abs(W_flags)) + 1e-8
    Wq = jnp.clip(jnp.round(W_flags / scale), -1.0, 1.0) * scale
    wgp, wg7 = W_gate[:8], _pad_weight7(W_gate[8:D_EMB])
    wgf = Wq.T @ W_gate[D_EMB:]
    bg = (b_gate + b_flags @ W_gate[D_EMB:]).reshape(1, D_OUT)
    wpp, wp7 = W_proj[:8], _pad_weight7(W_proj[8:D_EMB])
    wpf = Wq.T @ W_proj[D_EMB:]
    bp = (b_proj + b_flags @ W_proj[D_EMB:]).reshape(1, D_OUT)

    ph, c7 = _sc_gather2(idx_ph, idx7, E_ph, tab7)
    return _tc_fuse(ph, c7.reshape(N, D7), flags2d,
                    wgp, wg7, wgf, bg, wpp, wp7, wpf, bp,
                    gamma.reshape(1, D_OUT), beta.reshape(1, D_OUT))
